# baseline (speedup 1.0000x reference)
import jax
import jax.numpy as jnp
from jax.experimental import pallas as pl
from jax.experimental.pallas import tpu as pltpu

N = 50000
F = 128
T = 32
S = 32
B = 128

TN = 2000
GRID = N // TN


def _ect_kernel(x_ref, b_ref, v_ref, lin_ref, ones_ref, o_ref, c_ref,
                sc_ref):
    i = pl.program_id(0)

    @pl.when(i == 0)
    def _init():
        o_ref[...] = jnp.zeros_like(o_ref)
        c_ref[...] = jnp.zeros_like(c_ref)
        sc_ref[1] = jnp.zeros((TN, S * T), jnp.bfloat16)

    pa = i % 2
    nh = jnp.dot(x_ref[...], v_ref[...], preferred_element_type=jnp.float32)
    nh16 = nh.astype(jnp.bfloat16)
    w = lin_ref[0:1, :] + jnp.tile(nh16, (1, S))
    sc_ref[pa] = jnp.tanh(w)

    bid = b_ref[0]
    iota_b = jax.lax.broadcasted_iota(jnp.int32, (B, TN), 0)
    half = jnp.where(i > 0, 0.5, 0.0)
    oh8 = jnp.where(iota_b == bid, half, 0.0).astype(jnp.float8_e4m3fn)
    th8 = sc_ref[(i + 1) % 2].astype(jnp.float8_e4m3fn)
    part = jnp.dot(oh8, th8, preferred_element_type=jnp.float32)
    cnt = jnp.dot(oh8, ones_ref[...],
                  preferred_element_type=jnp.float32)
    o_ref[...] += part
    c_ref[...] += cnt


def kernel(x, batch, v, lin):
    batch3d = batch.reshape(GRID, 1, TN)
    v100 = v * (-100.0)
    lin100 = jnp.broadcast_to(
        (100.0 * jnp.repeat(lin.reshape(-1), T)).reshape(1, S * T), (8, S * T)
    ).astype(jnp.bfloat16)
    ones8 = jnp.ones((TN, 128), jnp.float8_e4m3fn)
    raw, cnt_half = pl.pallas_call(
        _ect_kernel,
        grid=(GRID + 1,),
        in_specs=[
            pl.BlockSpec((TN, F), lambda i: (jnp.minimum(i, GRID - 1), 0)),
            pl.BlockSpec((1, 1, TN),
                         lambda i: (jnp.maximum(i - 1, 0), 0, 0)),
            pl.BlockSpec((F, T), lambda i: (0, 0)),
            pl.BlockSpec((8, S * T), lambda i: (0, 0)),
            pl.BlockSpec((TN, 128), lambda i: (0, 0)),
        ],
        out_specs=[
            pl.BlockSpec((B, S * T), lambda i: (0, 0)),
            pl.BlockSpec((B, 128), lambda i: (0, 0)),
        ],
        out_shape=[
            jax.ShapeDtypeStruct((B, S * T), jnp.float32),
            jax.ShapeDtypeStruct((B, 128), jnp.float32),
        ],
        scratch_shapes=[pltpu.VMEM((2, TN, S * T), jnp.bfloat16)],
    )(x, batch3d, v100, lin100, ones8)
    out2d = raw + cnt_half[:, 0:1]
    return out2d.reshape(B, S, T)


# speedup vs baseline: 1.4140x; 1.4140x over previous
import jax
import jax.numpy as jnp
from jax.experimental import pallas as pl

N = 50000
F = 128
T = 32
S = 32
B = 128

TN = 5000
GRID = N // TN
NC = 4
CN = TN // NC


def _ect_kernel(x_ref, b_ref, v_ref, lin_ref, o_ref):
    i = pl.program_id(0)
    parts = []
    for c in range(NC):
        rows = pl.ds(c * CN, CN)
        nh = jnp.dot(x_ref[rows, :], v_ref[...],
                     preferred_element_type=jnp.float32)
        nh16 = nh.astype(jnp.bfloat16)
        w = lin_ref[0:1, :] + jnp.tile(nh16, (1, S))
        ecc = (1.0 + jnp.tanh(w)).astype(jnp.float8_e4m3fn)
        bid = b_ref[0, 0:1, rows]
        iota_b = jax.lax.broadcasted_iota(jnp.int32, (B, CN), 0)
        oh8 = jnp.where(iota_b == bid, 0.5, 0.0).astype(jnp.float8_e4m3fn)
        parts.append(jnp.dot(oh8, ecc, preferred_element_type=jnp.float32))
    part = (parts[0] + parts[1]) + (parts[2] + parts[3])

    @pl.when(i == 0)
    def _init():
        o_ref[...] = part

    @pl.when(i > 0)
    def _acc():
        o_ref[...] += part


def kernel(x, batch, v, lin):
    batch3d = batch.reshape(GRID, 1, TN)
    v100 = v * (-100.0)
    lin100 = jnp.broadcast_to(
        (100.0 * jnp.repeat(lin.reshape(-1), T)).reshape(1, S * T), (8, S * T)
    ).astype(jnp.bfloat16)
    out2d = pl.pallas_call(
        _ect_kernel,
        grid=(GRID,),
        in_specs=[
            pl.BlockSpec((TN, F), lambda i: (i, 0)),
            pl.BlockSpec((1, 1, TN), lambda i: (i, 0, 0)),
            pl.BlockSpec((F, T), lambda i: (0, 0)),
            pl.BlockSpec((8, S * T), lambda i: (0, 0)),
        ],
        out_specs=pl.BlockSpec((B, S * T), lambda i: (0, 0)),
        out_shape=jax.ShapeDtypeStruct((B, S * T), jnp.float32),
    )(x, batch3d, v100, lin100)
    return out2d.reshape(B, S, T)


# speedup vs baseline: 1.5305x; 1.0824x over previous
import jax
import jax.numpy as jnp
from jax.experimental import pallas as pl

N = 50000
F = 128
T = 32
S = 32
B = 128

TN = 10000
GRID = N // TN
NC = 2
CN = TN // NC


def _ect_kernel(x_ref, b_ref, v_ref, lin_ref, o_ref):
    i = pl.program_id(0)
    parts = []
    for c in range(NC):
        rows = pl.ds(c * CN, CN)
        nh = jnp.dot(x_ref[rows, :], v_ref[...],
                     preferred_element_type=jnp.float32)
        nh16 = nh.astype(jnp.bfloat16)
        w = lin_ref[0:1, :] + jnp.tile(nh16, (1, S))
        ecc = (1.0 + jnp.tanh(w)).astype(jnp.float8_e4m3fn)
        bid = b_ref[0, 0:1, rows]
        iota_b = jax.lax.broadcasted_iota(jnp.int32, (B, CN), 0)
        oh8 = jnp.where(iota_b == bid, 0.5, 0.0).astype(jnp.float8_e4m3fn)
        parts.append(jnp.dot(oh8, ecc, preferred_element_type=jnp.float32))
    part = sum(parts[1:], parts[0])

    @pl.when(i == 0)
    def _init():
        o_ref[...] = part

    @pl.when(i > 0)
    def _acc():
        o_ref[...] += part


def kernel(x, batch, v, lin):
    batch3d = batch.reshape(GRID, 1, TN)
    v100 = v * (-100.0)
    lin100 = jnp.broadcast_to(
        (100.0 * jnp.repeat(lin.reshape(-1), T)).reshape(1, S * T), (8, S * T)
    ).astype(jnp.bfloat16)
    out2d = pl.pallas_call(
        _ect_kernel,
        grid=(GRID,),
        in_specs=[
            pl.BlockSpec((TN, F), lambda i: (i, 0)),
            pl.BlockSpec((1, 1, TN), lambda i: (i, 0, 0)),
            pl.BlockSpec((F, T), lambda i: (0, 0)),
            pl.BlockSpec((8, S * T), lambda i: (0, 0)),
        ],
        out_specs=pl.BlockSpec((B, S * T), lambda i: (0, 0)),
        out_shape=jax.ShapeDtypeStruct((B, S * T), jnp.float32),
    )(x, batch3d, v100, lin100)
    return out2d.reshape(B, S, T)
